# Initial kernel scaffold; baseline (speedup 1.0000x reference)
import jax
import jax.numpy as jnp
from jax.experimental import pallas as pl


def kernel(x):
    raise NotImplementedError("write your pallas kernel here")


# speedup vs baseline: 3.0360x; 3.0360x over previous
import functools

import jax
import jax.numpy as jnp
from jax import lax
from jax.experimental import pallas as pl
from jax.experimental.pallas import tpu as pltpu
from jax.experimental.pallas import tpu_sc as plsc

B, F, T = 4096, 128, 100
N = B * F
NC, NS, L = 2, 16, 16
NW = NC * NS
ROWS_W = N // NW
R = 256
NCHUNK = ROWS_W // R
CW = R * T


def _spike_times(xv):
    xn = jnp.minimum(jnp.maximum((xv + 1.0) * 0.5, 0.0), 1.0)
    v = (1.0 - xn) * 99.0
    fv = v + 0.5
    ti = fv.astype(jnp.int32)
    tie = ti.astype(jnp.float32) == fv
    odd = (ti & 1) == 1
    ti = ti - jnp.where(tie & odd, 1, 0)
    return jnp.minimum(jnp.maximum(ti, 0), T - 1)


@functools.partial(
    pl.kernel,
    out_type=jax.ShapeDtypeStruct((N * T,), jnp.float32),
    mesh=plsc.VectorSubcoreMesh(core_axis_name="c", subcore_axis_name="s"),
    compiler_params=pltpu.CompilerParams(needs_layout_passes=False),
    scratch_types=[
        pltpu.VMEM((ROWS_W,), jnp.float32),
        pltpu.VMEM((CW,), jnp.float32),
        pltpu.VMEM((CW,), jnp.float32),
        pltpu.VMEM((R,), jnp.int32),
        pltpu.VMEM((R,), jnp.int32),
        pltpu.SemaphoreType.DMA,
        pltpu.SemaphoreType.DMA,
    ],
)
def _encode(x_hbm, out_hbm, xbuf, ob0, ob1, ib0, ib1, sem0, sem1):
    wid = lax.axis_index("s") * NC + lax.axis_index("c")
    row0 = wid * ROWS_W
    pltpu.sync_copy(x_hbm.at[pl.ds(row0, ROWS_W)], xbuf)

    zeros = jnp.zeros((L,), jnp.float32)
    ones = jnp.full((L,), 1.0, jnp.float32)
    lane_rows = lax.iota(jnp.int32, L) * T

    def _zero_init(i, _):
        ob0[pl.ds(i * L, L)] = zeros
        ob1[pl.ds(i * L, L)] = zeros
        return 0

    lax.fori_loop(0, CW // L, _zero_init, 0)

    obufs, ibufs, sems = (ob0, ob1), (ib0, ib1), (sem0, sem1)
    copies = [None] * NCHUNK
    for c in range(NCHUNK):
        p = c & 1
        ob, ib = obufs[p], ibufs[p]
        if c >= 2:
            copies[c - 2].wait()

            def _rezero(j, _, ob=ob, ib=ib):
                idx = ib[pl.ds(j * L, L)]
                plsc.store_scatter(ob, [idx], zeros)
                return 0

            lax.fori_loop(0, R // L, _rezero, 0)

        def _set_ones(j, _, ob=ob, ib=ib, c=c):
            xv = xbuf[pl.ds(c * R + j * L, L)]
            idx = j * (L * T) + lane_rows + _spike_times(xv)
            plsc.store_scatter(ob, [idx], ones)
            ib[pl.ds(j * L, L)] = idx
            return 0

        lax.fori_loop(0, R // L, _set_ones, 0)
        dst = out_hbm.at[pl.ds((row0 + c * R) * T, CW)]
        copies[c] = pltpu.async_copy(ob, dst, sems[p])

    copies[NCHUNK - 2].wait()
    copies[NCHUNK - 1].wait()


def kernel(x):
    return _encode(x.reshape(N)).reshape(B, F, T)


# speedup vs baseline: 6.8235x; 2.2476x over previous
import functools

import jax
import jax.numpy as jnp
from jax import lax
from jax.experimental import pallas as pl
from jax.experimental.pallas import tpu as pltpu
from jax.experimental.pallas import tpu_sc as plsc

B, F, T = 4096, 128, 100
N = B * F
NC, NS, L = 2, 16, 16
NW = NC * NS
PLANES_W = B // NW
ROWS_W = N // NW
NB = 2
R = NB * F
NCHUNK = PLANES_W // NB
CW = R * T


def _spike_times(xv):
    xn = jnp.minimum(jnp.maximum((xv + 1.0) * 0.5, 0.0), 1.0)
    v = (1.0 - xn) * 99.0
    fv = v + 0.5
    ti = fv.astype(jnp.int32)
    tie = ti.astype(jnp.float32) == fv
    odd = (ti & 1) == 1
    ti = ti - jnp.where(tie & odd, 1, 0)
    return jnp.minimum(jnp.maximum(ti, 0), T - 1)


@functools.partial(
    pl.kernel,
    out_type=jax.ShapeDtypeStruct((B, F, T), jnp.float32),
    mesh=plsc.VectorSubcoreMesh(core_axis_name="c", subcore_axis_name="s"),
    compiler_params=pltpu.CompilerParams(needs_layout_passes=False),
    scratch_types=[
        pltpu.VMEM((ROWS_W,), jnp.float32),
        pltpu.VMEM((NB, F, T), jnp.float32),
        pltpu.VMEM((NB, F, T), jnp.float32),
        pltpu.VMEM((R,), jnp.int32),
        pltpu.VMEM((R,), jnp.int32),
        pltpu.SemaphoreType.DMA,
        pltpu.SemaphoreType.DMA,
    ],
)
def _encode(x_hbm, out_hbm, xbuf, ob0, ob1, ib0, ib1, sem0, sem1):
    wid = lax.axis_index("s") * NC + lax.axis_index("c")
    row0 = wid * ROWS_W
    plane0 = wid * PLANES_W
    pltpu.sync_copy(x_hbm.at[pl.ds(row0, ROWS_W)], xbuf)

    zeros = jnp.zeros((L,), jnp.float32)
    ones = jnp.full((L,), 1.0, jnp.float32)
    lanes = lax.iota(jnp.int32, L)

    def _zero_init(i, _):
        q = i * L + lanes
        qb = q // (F * T)
        qr = q % (F * T)
        qf = qr // T
        qt = qr % T
        plsc.store_scatter(ob0, [qb, qf, qt], zeros)
        plsc.store_scatter(ob1, [qb, qf, qt], zeros)
        return 0

    lax.fori_loop(0, CW // L, _zero_init, 0)

    obufs, ibufs, sems = (ob0, ob1), (ib0, ib1), (sem0, sem1)
    copies = [None] * NCHUNK
    for c in range(NCHUNK):
        p = c & 1
        ob, ib = obufs[p], ibufs[p]
        if c >= 2:
            copies[c - 2].wait()

            def _rezero(j, _, ob=ob, ib=ib):
                idx_b = jnp.full((L,), 0, jnp.int32) + j // 8
                idx_f = (j % 8) * L + lanes
                idx_t = ib[pl.ds(j * L, L)]
                plsc.store_scatter(ob, [idx_b, idx_f, idx_t], zeros)
                return 0

            lax.fori_loop(0, R // L, _rezero, 0)

        def _set_ones(j, _, ob=ob, ib=ib, c=c):
            xv = xbuf[pl.ds(c * R + j * L, L)]
            ti = _spike_times(xv)
            idx_b = jnp.full((L,), 0, jnp.int32) + j // 8
            idx_f = (j % 8) * L + lanes
            plsc.store_scatter(ob, [idx_b, idx_f, ti], ones)
            ib[pl.ds(j * L, L)] = ti
            return 0

        lax.fori_loop(0, R // L, _set_ones, 0)
        dst = out_hbm.at[pl.ds(plane0 + c * NB, NB)]
        copies[c] = pltpu.async_copy(ob, dst, sems[p])

    copies[NCHUNK - 2].wait()
    copies[NCHUNK - 1].wait()


def kernel(x):
    return _encode(x.reshape(N))
